# baseline (speedup 1.0000x reference)
import functools

import jax
import jax.numpy as jnp
from jax import lax
from jax.experimental import pallas as pl
from jax.experimental.pallas import tpu as pltpu
from jax.experimental.pallas import tpu_sc as plsc

_NC = 2
_NS = 16
_NW = _NC * _NS

_D = 64
_CHUNK = 80
_WB = 400
_CPW = _WB // _CHUNK


def _make_gather(total):
    n_per_w = total // _NW
    n_chunks = n_per_w // _CHUNK

    mesh = plsc.VectorSubcoreMesh(
        core_axis_name="c", subcore_axis_name="s",
        num_cores=_NC, num_subcores=_NS)

    @functools.partial(
        pl.kernel,
        mesh=mesh,
        compiler_params=pltpu.CompilerParams(needs_layout_passes=False),
        out_type=jax.ShapeDtypeStruct((total // 2, 2 * _D), jnp.float32),
        scratch_types=[
            pltpu.VMEM((n_per_w,), jnp.int32),
            pltpu.VMEM((_CHUNK,), jnp.int32),
            pltpu.VMEM((_CHUNK,), jnp.int32),
            pltpu.VMEM((_CHUNK, 2 * _D), jnp.float32),
            pltpu.VMEM((_CHUNK, 2 * _D), jnp.float32),
            pltpu.VMEM((_WB // 2, 2 * _D), jnp.float32),
            pltpu.SemaphoreType.DMA,
            pltpu.SemaphoreType.DMA,
        ],
    )
    def gather(table_hbm, idx_hbm, out_hbm, idx_v, pid_a, pid_b,
               pairs_a, pairs_b, rows_v, sem_a, sem_b):
        wid = lax.axis_index("s") * _NC + lax.axis_index("c")
        base = pl.multiple_of(wid * n_per_w, 128)
        pltpu.sync_copy(idx_hbm.at[pl.ds(base, n_per_w)], idx_v)

        pids = (pid_a, pid_b)
        pairs = (pairs_a, pairs_b)
        sems = (sem_a, sem_b)

        def issue(j, slot):
            off = j * _CHUNK
            for i in range(_CHUNK // 16):
                v = idx_v[pl.ds(off + i * 16, 16)]
                pids[slot][pl.ds(i * 16, 16)] = lax.shift_right_logical(v, 1)
            pltpu.async_copy(table_hbm.at[pids[slot]], pairs[slot],
                             sems[slot])

        def extract(j, slot):
            off = j * _CHUNK
            lanes = lax.iota(jnp.int32, 16)
            gbase = lax.rem(lax.mul(j, _CHUNK), _WB)

            def row_body(r, _):
                rv = jnp.broadcast_to(r, (16,)).astype(jnp.int32)
                iv = plsc.load_gather(idx_v, [rv + off])
                halfc = lax.mul(lax.bitwise_and(iv, 1), _D)
                g = gbase + r
                dst_r = g // 2
                dst_c = lax.rem(g, 2) * _D
                for colg in range(_D // 16):
                    cvec = halfc + (colg * 16) + lanes
                    vals = plsc.load_gather(pairs[slot], [rv, cvec])
                    rows_v[dst_r, pl.ds(dst_c + colg * 16, 16)] = vals
                return 0

            lax.fori_loop(0, _CHUNK, row_body, 0)

        def flush(j):
            wb_i = j // _CPW
            r0 = pl.multiple_of((base + wb_i * _WB) // 2, 8)
            pltpu.sync_copy(rows_v, out_hbm.at[pl.ds(r0, _WB // 2)])

        issue(0, 0)

        def pair_body(m, carry):
            j0 = m * 2
            issue(j0 + 1, 1)
            pltpu.make_async_copy(table_hbm.at[pids[0]], pairs[0],
                                  sems[0]).wait()
            extract(j0, 0)

            @pl.when(lax.rem(j0, _CPW) == _CPW - 1)
            def _():
                flush(j0)

            @pl.when(j0 + 2 < n_chunks)
            def _():
                issue(j0 + 2, 0)

            pltpu.make_async_copy(table_hbm.at[pids[1]], pairs[1],
                                  sems[1]).wait()
            extract(j0 + 1, 1)

            @pl.when(lax.rem(j0 + 1, _CPW) == _CPW - 1)
            def _():
                flush(j0 + 1)
            return 0

        lax.fori_loop(0, n_chunks // 2, pair_body, 0)

    return gather


def kernel(x, index, latents):
    b, h = index.shape
    num_rows, d = latents.shape
    table128 = latents.reshape(num_rows // 2, 2 * d)
    idx_flat = index.reshape(b * h).astype(jnp.int32)
    out128 = _make_gather(b * h)(table128, idx_flat)
    return out128.reshape(b, h, d)


# speedup vs baseline: 1.1718x; 1.1718x over previous
import functools

import jax
import jax.numpy as jnp
from jax import lax
from jax.experimental import pallas as pl
from jax.experimental.pallas import tpu as pltpu
from jax.experimental.pallas import tpu_sc as plsc

_NC = 2
_NS = 16
_NW = _NC * _NS

_B = 4096
_H = 50
_D = 64
_CHUNK = 800


def _make_gather(total, dtype):
    n_per_w = total // _NW
    n_chunks = n_per_w // _CHUNK
    b_per_chunk = _CHUNK // _H

    mesh = plsc.VectorSubcoreMesh(
        core_axis_name="c", subcore_axis_name="s",
        num_cores=_NC, num_subcores=_NS)

    @functools.partial(
        pl.kernel,
        mesh=mesh,
        compiler_params=pltpu.CompilerParams(use_tc_tiling_on_sc=False),
        out_type=jax.ShapeDtypeStruct((_B, _H, _D), dtype),
        scratch_types=[
            pltpu.VMEM((n_per_w,), jnp.int32),
            pltpu.VMEM((_CHUNK, _D), dtype),
            pltpu.VMEM((_CHUNK, _D), dtype),
            pltpu.SemaphoreType.DMA,
            pltpu.SemaphoreType.DMA,
        ],
    )
    def gather(table_hbm, idx_hbm, out_hbm, idx_v, rows_a, rows_b,
               sem_a, sem_b):
        wid = lax.axis_index("s") * _NC + lax.axis_index("c")
        base = wid * n_per_w
        b_base = wid * (n_per_w // _H)
        pltpu.sync_copy(idx_hbm.at[pl.ds(base, n_per_w)], idx_v)
        bufs = (rows_a, rows_b)
        sems = (sem_a, sem_b)

        def drain(j):
            slot = j % 2
            pltpu.make_async_copy(
                table_hbm.at[idx_v.at[pl.ds(j * _CHUNK, _CHUNK)]],
                bufs[slot], sems[slot]).wait()
            b0 = b_base + j * b_per_chunk
            for k in range(b_per_chunk):
                pltpu.sync_copy(bufs[slot].at[pl.ds(k * _H, _H)],
                                out_hbm.at[b0 + k])

        pltpu.async_copy(
            table_hbm.at[idx_v.at[pl.ds(0, _CHUNK)]], bufs[0], sems[0])
        for j in range(1, n_chunks):
            pltpu.async_copy(
                table_hbm.at[idx_v.at[pl.ds(j * _CHUNK, _CHUNK)]],
                bufs[j % 2], sems[j % 2])
            drain(j - 1)
        drain(n_chunks - 1)

    return gather


def kernel(x, index, latents):
    b, h = index.shape
    idx_flat = index.reshape(b * h).astype(jnp.int32)
    return _make_gather(b * h, latents.dtype)(latents, idx_flat)
